# Initial kernel scaffold; baseline (speedup 1.0000x reference)
import jax
import jax.numpy as jnp
from jax.experimental import pallas as pl


def kernel(obj_pred_0, cls_pred_0, reg_pred_0, anchors_0, obj_pred_1, cls_pred_1, reg_pred_1, anchors_1, obj_pred_2, cls_pred_2, reg_pred_2, anchors_2):
    raise NotImplementedError("write your pallas kernel here")


# speedup vs baseline: 1.4517x; 1.4517x over previous
import jax
import jax.numpy as jnp
from jax.experimental import pallas as pl
from jax.experimental.pallas import tpu as pltpu

_NC = 20
_K = 100
_KP = 128
_CONF = 0.01
_NMS_T = 0.5
_STRIDES = (8.0, 16.0, 32.0)
_IMAX = 2147483647


def _process_level(l, obj_ref, cls_ref, reg_ref, anc_ref):
    A = obj_ref.shape[0]
    rowmax = (jax.nn.sigmoid(obj_ref[:, 0])
              * jax.nn.sigmoid(jnp.max(cls_ref[:], axis=1)))
    iota_a = jax.lax.broadcasted_iota(jnp.int32, (A, 1), 0)[:, 0]
    iota_kp = jax.lax.broadcasted_iota(jnp.int32, (_KP, 1), 0)[:, 0]

    def amax_body(k, carry):
        v, acc = carry
        m = jnp.max(v)
        cand = v == m
        idx = jnp.min(jnp.where(cand, iota_a, _IMAX))
        acc = jnp.where(iota_kp == k, idx, acc)
        return jnp.where(iota_a == idx, -1.0, v), acc

    _, aidx = jax.lax.fori_loop(
        0, _K, amax_body, (rowmax, jnp.full((_KP,), -1, jnp.int32)))

    onehot = (jax.lax.broadcasted_iota(jnp.int32, (_KP, A), 1)
              == aidx[:, None]).astype(jnp.float32)
    gdata = jnp.concatenate(
        [obj_ref[:], cls_ref[:], anc_ref[:], reg_ref[:]], axis=1)
    g = jnp.dot(onehot, gdata, preferred_element_type=jnp.float32,
                precision=jax.lax.Precision.HIGHEST)
    g_obj, g_cls = g[:, 0:1], g[:, 1:1 + _NC]
    g_anc, g_reg = g[:, 21:25], g[:, 25:29]
    g_scores = jax.nn.sigmoid(g_obj) * jax.nn.sigmoid(g_cls)

    iota_c = jax.lax.broadcasted_iota(jnp.int32, (_KP, _NC), 1)
    row_iota = jax.lax.broadcasted_iota(jnp.int32, (_KP, _NC), 0)
    flat = aidx[:, None] * _NC + iota_c
    ev0 = jnp.where(row_iota < _K, g_scores, -1.0)

    def emax_body(k, carry):
        ev, svals, labs, rows = carry
        m = jnp.max(ev)
        cand = ev == m
        sel_flat = jnp.min(jnp.where(cand, flat, _IMAX))
        pm = cand & (flat == sel_flat)
        row = jnp.min(jnp.where(pm, row_iota, _IMAX))
        sel = iota_kp == k
        svals = jnp.where(sel, m, svals)
        labs = jnp.where(sel, sel_flat % _NC, labs)
        rows = jnp.where(sel, row, rows)
        return jnp.where(pm, -1.0, ev), svals, labs, rows

    _, svals, labs, rows = jax.lax.fori_loop(
        0, _K, emax_body,
        (ev0, jnp.full((_KP,), -1.0, jnp.float32),
         jnp.zeros((_KP,), jnp.int32), jnp.zeros((_KP,), jnp.int32)))

    ctr = (jax.nn.sigmoid(g_reg[:, :2]) * 3.0 - 1.5 + g_anc[:, :2]) * _STRIDES[l]
    wh = jnp.exp(g_reg[:, 2:]) * g_anc[:, 2:]
    boxes = jnp.concatenate([ctr - wh * 0.5, ctr + wh * 0.5], axis=1)

    onehot2 = (jax.lax.broadcasted_iota(jnp.int32, (_KP, _KP), 1)
               == rows[:, None]).astype(jnp.float32)
    boxes_sel = jnp.dot(onehot2, boxes, preferred_element_type=jnp.float32,
                        precision=jax.lax.Precision.HIGHEST)
    return svals, labs, boxes_sel


def _nms_body(obj0, cls0, reg0, anc0, obj1, cls1, reg1, anc1,
              obj2, cls2, reg2, anc2,
              box_out, score_out, label_out, keep_out):
    sv0, lb0, bx0 = _process_level(0, obj0, cls0, reg0, anc0)
    sv1, lb1, bx1 = _process_level(1, obj1, cls1, reg1, anc1)
    sv2, lb2, bx2 = _process_level(2, obj2, cls2, reg2, anc2)

    N = 3 * _KP
    s = jnp.concatenate([sv0, sv1, sv2])
    lab = jnp.concatenate([lb0, lb1, lb2])
    b = jnp.concatenate([bx0, bx1, bx2], axis=0)
    valid = s > _CONF

    boff = b + lab.astype(jnp.float32)[:, None] * 100000.0
    x1, y1, x2, y2 = boff[:, 0], boff[:, 1], boff[:, 2], boff[:, 3]
    areas = (x2 - x1) * (y2 - y1)
    xx1 = jnp.maximum(x1[:, None], x1[None, :])
    yy1 = jnp.maximum(y1[:, None], y1[None, :])
    xx2 = jnp.minimum(x2[:, None], x2[None, :])
    yy2 = jnp.minimum(y2[:, None], y2[None, :])
    w = jnp.maximum(1e-10, xx2 - xx1)
    h = jnp.maximum(1e-10, yy2 - yy1)
    inter = w * h
    iou = inter / (areas[:, None] + areas[None, :] - inter)

    ii = jax.lax.broadcasted_iota(jnp.int32, (N, N), 0)
    jj = jax.lax.broadcasted_iota(jnp.int32, (N, N), 1)
    prec = (s[None, :] > s[:, None]) | ((s[None, :] == s[:, None]) & (jj < ii))
    sup = ((iou > _NMS_T) & prec).astype(jnp.float32)
    validf = valid.astype(jnp.float32)

    def cond(state):
        it, _, changed = state
        return changed & (it <= N + 1)

    def body(state):
        it, keep, _ = state
        sup_any = jnp.max(sup * keep[None, :], axis=1)
        keep_new = validf * (1.0 - jnp.minimum(sup_any, 1.0))
        changed = jnp.any(keep_new != keep)
        return it + 1, keep_new, changed

    _, keepf, _ = jax.lax.while_loop(
        cond, body, (jnp.int32(0), validf, jnp.bool_(True)))

    box_out[:, :] = b * keepf[:, None]
    score_out[:] = s * keepf
    label_out[:] = lab
    keep_out[:] = (keepf > 0.5).astype(jnp.int32)


def kernel(obj_pred_0, cls_pred_0, reg_pred_0, anchors_0,
           obj_pred_1, cls_pred_1, reg_pred_1, anchors_1,
           obj_pred_2, cls_pred_2, reg_pred_2, anchors_2):
    N = 3 * _KP
    bboxes, scores, labels, keep = pl.pallas_call(
        _nms_body,
        out_shape=[
            jax.ShapeDtypeStruct((N, 4), jnp.float32),
            jax.ShapeDtypeStruct((N,), jnp.float32),
            jax.ShapeDtypeStruct((N,), jnp.int32),
            jax.ShapeDtypeStruct((N,), jnp.int32),
        ],
        compiler_params=pltpu.CompilerParams(
            vmem_limit_bytes=100 * 1024 * 1024),
    )(obj_pred_0, cls_pred_0, reg_pred_0, anchors_0,
      obj_pred_1, cls_pred_1, reg_pred_1, anchors_1,
      obj_pred_2, cls_pred_2, reg_pred_2, anchors_2)
    bboxes = bboxes.reshape(3, _KP, 4)[:, :_K].reshape(3 * _K, 4)
    scores = scores.reshape(3, _KP)[:, :_K].reshape(3 * _K)
    labels = labels.reshape(3, _KP)[:, :_K].reshape(3 * _K)
    keep = keep.reshape(3, _KP)[:, :_K].reshape(3 * _K)
    return bboxes, scores, labels, keep.astype(bool)


# speedup vs baseline: 25.0460x; 17.2526x over previous
import jax
import jax.numpy as jnp
from jax.experimental import pallas as pl
from jax.experimental.pallas import tpu as pltpu

_NC = 20
_NCP = 32
_K = 100
_KP = 128
_CONF = 0.01
_NMS_T = 0.5
_STRIDES = (8.0, 16.0, 32.0)
_IMAX = 2147483647
_LEVEL_ANCHORS = (12288, 3072, 768)


def _pre(sa, fa, sb, fb):
    return (sa > sb) | ((sa == sb) & (fa < fb))


def _roll(x, sh, axis):
    n = x.shape[axis]
    sh %= n
    if sh == 0:
        return x
    if axis == 0:
        return jnp.concatenate([x[n - sh:], x[:n - sh]], axis=0)
    return pltpu.roll(x, sh, axis)


def _bitonic_desc(s, f, p):
    R, W = s.shape
    M = R * W
    L = M.bit_length() - 1
    lw = W.bit_length() - 1
    ir = jax.lax.broadcasted_iota(jnp.int32, (R, W), 0)
    il = jax.lax.broadcasted_iota(jnp.int32, (R, W), 1)
    idx = ir * W + il
    for ke in range(1, L + 1):
        for je in range(ke - 1, -1, -1):
            d = 1 << je
            hi = ((idx >> je) & 1) == 1
            rev = ((idx >> ke) & 1) == 1
            if d >= W:
                axis, sh = 0, d >> lw
            else:
                axis, sh = 1, d
            ps = jnp.where(hi, _roll(s, sh, axis), _roll(s, -sh, axis))
            pf = jnp.where(hi, _roll(f, sh, axis), _roll(f, -sh, axis))
            pp = jnp.where(hi, _roll(p, sh, axis), _roll(p, -sh, axis))
            lo_s = jnp.where(hi, ps, s)
            hi_s = jnp.where(hi, s, ps)
            lo_f = jnp.where(hi, pf, f)
            hi_f = jnp.where(hi, f, pf)
            swap = ((rev & _pre(lo_s, lo_f, hi_s, hi_f))
                    | (~rev & _pre(hi_s, hi_f, lo_s, lo_f)))
            s = jnp.where(swap, ps, s)
            f = jnp.where(swap, pf, f)
            p = jnp.where(swap, pp, p)
    return s, f, p


def _process_level(l, obj2d_ref, cls3_ref, dataT_ref):
    A = _LEVEL_ANCHORS[l]
    R = A // 128
    RP = 1 << ((R - 1).bit_length())
    mx = cls3_ref[0:R]
    for c in range(1, _NC):
        mx = jnp.maximum(mx, cls3_ref[c * R:(c + 1) * R])
    rowmax = jax.nn.sigmoid(obj2d_ref[:]) * jax.nn.sigmoid(mx)
    rowmax = jnp.concatenate(
        [rowmax, jnp.full((RP - R, 128), -1.0, jnp.float32)], axis=0)
    ir = jax.lax.broadcasted_iota(jnp.int32, (RP, 128), 0)
    il = jax.lax.broadcasted_iota(jnp.int32, (RP, 128), 1)
    aidx_grid = ir * 128 + il

    _, aidx_sorted, _ = _bitonic_desc(rowmax, aidx_grid, aidx_grid)
    aidx = aidx_sorted[0]

    onehotT = (jax.lax.broadcasted_iota(jnp.int32, (A, _KP), 0)
               == aidx[None, :]).astype(jnp.float32)
    gT = jnp.dot(dataT_ref[:], onehotT, preferred_element_type=jnp.float32,
                 precision=jax.lax.Precision.HIGHEST)
    g_objT, g_clsT = gT[0:1], gT[1:1 + _NC]
    g_ancT, g_regT = gT[21:25], gT[25:29]
    g_scoresT = jax.nn.sigmoid(g_objT) * jax.nn.sigmoid(g_clsT)

    ic = jax.lax.broadcasted_iota(jnp.int32, (_NCP, _KP), 0)
    ie = jax.lax.broadcasted_iota(jnp.int32, (_NCP, _KP), 1)
    evT = jnp.concatenate(
        [g_scoresT, jnp.full((_NCP - _NC, _KP), -1.0, jnp.float32)], axis=0)
    evT = jnp.where((ic < _NC) & (ie < _K), evT, -1.0)
    flatT = jnp.where(ic < _NC, aidx[None, :] * _NC + ic, _IMAX)
    ss, sf, sp = _bitonic_desc(evT, flatT, ie)
    pos128 = jax.lax.broadcasted_iota(jnp.int32, (1, _KP), 1)[0]
    svals = jnp.where(pos128 < _K, ss[0], -1.0)
    labs = sf[0] % _NC
    rsel = sp[0]

    ctrT = ((jax.nn.sigmoid(g_regT[:2]) * 3.0 - 1.5 + g_ancT[:2])
            * _STRIDES[l])
    whT = jnp.exp(g_regT[2:]) * g_ancT[2:]
    boxesT = jnp.concatenate([ctrT - whT * 0.5, ctrT + whT * 0.5], axis=0)

    onehot2 = (jax.lax.broadcasted_iota(jnp.int32, (_KP, _KP), 0)
               == rsel[None, :]).astype(jnp.float32)
    boxes_selT = jnp.dot(boxesT, onehot2, preferred_element_type=jnp.float32,
                         precision=jax.lax.Precision.HIGHEST)
    return svals, labs, boxes_selT


def _nms_body(obj0, cls0, dat0, obj1, cls1, dat1, obj2, cls2, dat2,
              box_out, score_out, label_out, keep_out):
    sv0, lb0, bx0 = _process_level(0, obj0, cls0, dat0)
    sv1, lb1, bx1 = _process_level(1, obj1, cls1, dat1)
    sv2, lb2, bx2 = _process_level(2, obj2, cls2, dat2)

    N = 3 * _KP
    s = jnp.concatenate([sv0, sv1, sv2])
    lab = jnp.concatenate([lb0, lb1, lb2])
    bT = jnp.concatenate([bx0, bx1, bx2], axis=1)
    valid = s > _CONF

    boffT = bT + lab.astype(jnp.float32)[None, :] * 100000.0
    x1, y1, x2, y2 = boffT[0], boffT[1], boffT[2], boffT[3]
    areas = (x2 - x1) * (y2 - y1)
    xx1 = jnp.maximum(x1[:, None], x1[None, :])
    yy1 = jnp.maximum(y1[:, None], y1[None, :])
    xx2 = jnp.minimum(x2[:, None], x2[None, :])
    yy2 = jnp.minimum(y2[:, None], y2[None, :])
    w = jnp.maximum(1e-10, xx2 - xx1)
    h = jnp.maximum(1e-10, yy2 - yy1)
    inter = w * h
    iou = inter / (areas[:, None] + areas[None, :] - inter)

    ii = jax.lax.broadcasted_iota(jnp.int32, (N, N), 0)
    jj = jax.lax.broadcasted_iota(jnp.int32, (N, N), 1)
    prec = (s[None, :] > s[:, None]) | ((s[None, :] == s[:, None]) & (jj < ii))
    sup = ((iou > _NMS_T) & prec).astype(jnp.float32)
    validf = valid.astype(jnp.float32)

    def cond(state):
        it, _, changed = state
        return changed & (it <= N + 1)

    def body(state):
        it, keep, _ = state
        sup_any = jnp.max(sup * keep[None, :], axis=1)
        keep_new = validf * (1.0 - jnp.minimum(sup_any, 1.0))
        changed = jnp.any(keep_new != keep)
        return it + 1, keep_new, changed

    _, keepf, _ = jax.lax.while_loop(
        cond, body, (jnp.int32(0), validf, jnp.bool_(True)))

    box_out[:, :] = bT * keepf[None, :]
    score_out[:] = s * keepf
    label_out[:] = lab
    keep_out[:] = (keepf > 0.5).astype(jnp.int32)


def kernel(obj_pred_0, cls_pred_0, reg_pred_0, anchors_0,
           obj_pred_1, cls_pred_1, reg_pred_1, anchors_1,
           obj_pred_2, cls_pred_2, reg_pred_2, anchors_2):
    N = 3 * _KP
    ins = []
    for obj, cls, reg, anc in ((obj_pred_0, cls_pred_0, reg_pred_0, anchors_0),
                               (obj_pred_1, cls_pred_1, reg_pred_1, anchors_1),
                               (obj_pred_2, cls_pred_2, reg_pred_2, anchors_2)):
        A = obj.shape[0]
        ins.append(obj.reshape(A // 128, 128))
        ins.append(cls.T.reshape(_NC * (A // 128), 128))
        ins.append(jnp.concatenate([obj.T, cls.T, anc.T, reg.T], axis=0))
    boxT, scores, labels, keep = pl.pallas_call(
        _nms_body,
        out_shape=[
            jax.ShapeDtypeStruct((4, N), jnp.float32),
            jax.ShapeDtypeStruct((N,), jnp.float32),
            jax.ShapeDtypeStruct((N,), jnp.int32),
            jax.ShapeDtypeStruct((N,), jnp.int32),
        ],
        compiler_params=pltpu.CompilerParams(
            vmem_limit_bytes=100 * 1024 * 1024),
    )(*ins)
    bboxes = boxT.T.reshape(3, _KP, 4)[:, :_K].reshape(3 * _K, 4)
    scores = scores.reshape(3, _KP)[:, :_K].reshape(3 * _K)
    labels = labels.reshape(3, _KP)[:, :_K].reshape(3 * _K)
    keep = keep.reshape(3, _KP)[:, :_K].reshape(3 * _K)
    return bboxes, scores, labels, keep.astype(bool)
